# Initial kernel scaffold; baseline (speedup 1.0000x reference)
import functools

import jax
import jax.numpy as jnp
from jax import lax
from jax.experimental import pallas as pl
from jax.experimental.pallas import tpu as pltpu
from jax.experimental.pallas import tpu_sc as plsc

_NC = 2
_NS = 16
_NW = _NC * _NS
_L = 16


def _sc_scores(B, NNEG, D, center, context, neg_flat, W_center, W_context):
    b_per_w = B // _NW
    CB = 32
    NCH = b_per_w // CB
    NIDX = CB * NNEG
    NSTREAM = NIDX // 128

    mesh = plsc.VectorSubcoreMesh(core_axis_name="c", subcore_axis_name="s")

    @functools.partial(
        pl.kernel,
        out_type=[
            jax.ShapeDtypeStruct((B,), jnp.float32),
            jax.ShapeDtypeStruct((NNEG, B), jnp.float32),
        ],
        mesh=mesh,
        scratch_types=[
            pltpu.VMEM((CB,), jnp.int32),
            pltpu.VMEM((CB,), jnp.int32),
            pltpu.VMEM((NIDX,), jnp.int32),
            pltpu.VMEM((CB, D), jnp.float32),
            pltpu.VMEM((CB, D), jnp.float32),
            pltpu.VMEM((NIDX, D), jnp.float32),
            pltpu.VMEM((B // _NW,), jnp.float32),
            pltpu.VMEM((NNEG, B // _NW), jnp.float32),
            pltpu.SemaphoreType.DMA,
        ],
    )
    def scores_kernel(center_hbm, context_hbm, neg_hbm, wc_hbm, wx_hbm,
                      pos_hbm, negt_hbm,
                      cidx_v, xidx_v, nidx_v, crows_v, xrows_v, nrows_v,
                      posb_v, negb_v, sem_rows):
        wid = lax.axis_index("s") * _NC + lax.axis_index("c")
        base = pl.multiple_of(wid * b_per_w, b_per_w)

        def chunk_body(cb, carry):
            b0 = pl.multiple_of(base + cb * CB, CB)
            pltpu.sync_copy(center_hbm.at[pl.ds(b0, CB)], cidx_v)
            pltpu.sync_copy(context_hbm.at[pl.ds(b0, CB)], xidx_v)
            pltpu.sync_copy(
                neg_hbm.at[pl.ds(pl.multiple_of(b0 * NNEG, NIDX), NIDX)],
                nidx_v)
            copies = [pltpu.async_copy(wc_hbm.at[cidx_v], crows_v, sem_rows),
                      pltpu.async_copy(wx_hbm.at[xidx_v], xrows_v, sem_rows)]
            for j in range(NSTREAM):
                copies.append(pltpu.async_copy(
                    wx_hbm.at[nidx_v.at[pl.ds(j * 128, 128)]],
                    nrows_v.at[pl.ds(j * 128, 128), :], sem_rows))
            for c in copies:
                c.wait()

            for g in range(CB // _L):
                brel = lax.iota(jnp.int32, _L) + g * _L
                nrow = [brel * NNEG + n for n in range(NNEG)]

                def d_body(d, accs):
                    dcol = jnp.full((_L,), d, jnp.int32)
                    cvec = plsc.load_gather(crows_v, [brel, dcol])
                    xvec = plsc.load_gather(xrows_v, [brel, dcol])
                    acc_p = accs[0] + cvec * xvec
                    new_n = []
                    for n in range(NNEG):
                        nv = plsc.load_gather(nrows_v, [nrow[n], dcol])
                        new_n.append(accs[1 + n] + nv * cvec)
                    return (acc_p, *new_n)

                zero = jnp.zeros((_L,), jnp.float32)
                accs = lax.fori_loop(0, D, d_body, (zero,) * (1 + NNEG))
                off = cb * CB + g * _L
                posb_v[pl.ds(off, _L)] = accs[0]
                for n in range(NNEG):
                    negb_v[n, pl.ds(off, _L)] = accs[1 + n]
            return carry

        lax.fori_loop(0, NCH, chunk_body, 0)

        pltpu.sync_copy(posb_v, pos_hbm.at[pl.ds(base, b_per_w)])
        for n in range(NNEG):
            pltpu.sync_copy(negb_v.at[n], negt_hbm.at[n, pl.ds(base, b_per_w)])

    return scores_kernel(center, context, neg_flat, W_center, W_context)


def _loss_kernel(pos_ref, neg_ref, out_ref, *, inv_b):
    def logsig(x):
        return jnp.minimum(x, 0.0) - jnp.log1p(jnp.exp(-jnp.abs(x)))

    s_pos = jnp.sum(logsig(pos_ref[...]))
    s_neg = jnp.sum(logsig(-neg_ref[...]))
    out_ref[0, 0] = -(s_pos + s_neg) * inv_b


def kernel(center, context, negatives, W_center, W_context):
    B, NNEG = negatives.shape
    D = W_center.shape[1]
    pos, negt = _sc_scores(B, NNEG, D,
                           center.astype(jnp.int32),
                           context.astype(jnp.int32),
                           negatives.reshape(-1).astype(jnp.int32),
                           W_center, W_context)
    loss = pl.pallas_call(
        functools.partial(_loss_kernel, inv_b=1.0 / B),
        out_shape=jax.ShapeDtypeStruct((1, 1), jnp.float32),
    )(pos.reshape(B // 128, 128), negt.reshape(NNEG * B // 128, 128))
    return loss[0, 0]


# speedup vs baseline: 3.9309x; 3.9309x over previous
import functools

import jax
import jax.numpy as jnp
from jax import lax
from jax.experimental import pallas as pl
from jax.experimental.pallas import tpu as pltpu
from jax.experimental.pallas import tpu_sc as plsc

_NC = 2
_NS = 16
_NW = _NC * _NS
_L = 16


def _sc_scores(B, NNEG, D, center, context, neg_flat, W_center, W_context):
    b_per_w = B // _NW
    CB = 32
    NCH = b_per_w // CB
    NIDX = CB * NNEG
    NSTREAM = NIDX // 128

    mesh = plsc.VectorSubcoreMesh(core_axis_name="c", subcore_axis_name="s")

    @functools.partial(
        pl.kernel,
        out_type=[
            jax.ShapeDtypeStruct((B,), jnp.float32),
            jax.ShapeDtypeStruct((NNEG, B), jnp.float32),
        ],
        mesh=mesh,
        scratch_types=[
            pltpu.VMEM((CB,), jnp.int32),
            pltpu.VMEM((CB,), jnp.int32),
            pltpu.VMEM((NIDX,), jnp.int32),
            pltpu.VMEM((CB, D), jnp.float32),
            pltpu.VMEM((CB, D), jnp.float32),
            pltpu.VMEM((NIDX, D), jnp.float32),
            pltpu.VMEM((B // _NW,), jnp.float32),
            pltpu.VMEM((NNEG, B // _NW), jnp.float32),
            pltpu.SemaphoreType.DMA,
        ],
        compiler_params=pltpu.CompilerParams(needs_layout_passes=False,
                                             use_tc_tiling_on_sc=False),
    )
    def scores_kernel(center_hbm, context_hbm, neg_hbm, wc_hbm, wx_hbm,
                      pos_hbm, negt_hbm,
                      cidx_v, xidx_v, nidx_v, crows_v, xrows_v, nrows_v,
                      posb_v, negb_v, sem_rows):
        wid = lax.axis_index("s") * _NC + lax.axis_index("c")
        base = pl.multiple_of(wid * b_per_w, b_per_w)

        def chunk_body(cb, carry):
            b0 = pl.multiple_of(base + cb * CB, CB)
            pltpu.sync_copy(center_hbm.at[pl.ds(b0, CB)], cidx_v)
            pltpu.sync_copy(context_hbm.at[pl.ds(b0, CB)], xidx_v)
            pltpu.sync_copy(
                neg_hbm.at[pl.ds(pl.multiple_of(b0 * NNEG, NIDX), NIDX)],
                nidx_v)
            copies = [
                pltpu.async_copy(wc_hbm.at[cidx_v], crows_v, sem_rows),
                pltpu.async_copy(wx_hbm.at[xidx_v], xrows_v, sem_rows),
            ]
            for j in range(NSTREAM):
                copies.append(pltpu.async_copy(
                    wx_hbm.at[nidx_v.at[pl.ds(j * 128, 128)]],
                    nrows_v.at[pl.ds(j * 128, 128), :], sem_rows))
            for c in copies:
                c.wait()

            for g in range(CB // _L):
                brel = lax.iota(jnp.int32, _L) + g * _L
                nrow = [brel * NNEG + n for n in range(NNEG)]

                def d_body(d, accs):
                    dvec = jnp.full((_L,), d, jnp.int32)
                    cvec = plsc.load_gather(crows_v, [brel, dvec])
                    xvec = plsc.load_gather(xrows_v, [brel, dvec])
                    acc_p = accs[0] + cvec * xvec
                    new_n = []
                    for n in range(NNEG):
                        nv = plsc.load_gather(nrows_v, [nrow[n], dvec])
                        new_n.append(accs[1 + n] + nv * cvec)
                    return (acc_p, *new_n)

                zero = jnp.zeros((_L,), jnp.float32)
                accs = lax.fori_loop(0, D, d_body, (zero,) * (1 + NNEG))
                off = cb * CB + g * _L
                posb_v[pl.ds(off, _L)] = accs[0]
                for n in range(NNEG):
                    negb_v[n, pl.ds(off, _L)] = accs[1 + n]
            return carry

        lax.fori_loop(0, NCH, chunk_body, 0)

        pltpu.sync_copy(posb_v, pos_hbm.at[pl.ds(base, b_per_w)])
        for n in range(NNEG):
            pltpu.sync_copy(negb_v.at[n], negt_hbm.at[n, pl.ds(base, b_per_w)])

    return scores_kernel(center, context, neg_flat, W_center, W_context)


def _loss_kernel(pos_ref, neg_ref, out_ref, *, inv_b):
    def logsig(x):
        return jnp.minimum(x, 0.0) - jnp.log1p(jnp.exp(-jnp.abs(x)))

    s_pos = jnp.sum(logsig(pos_ref[...]))
    s_neg = jnp.sum(logsig(-neg_ref[...]))
    out_ref[...] = jnp.broadcast_to(-(s_pos + s_neg) * inv_b, (1, 1))


def kernel(center, context, negatives, W_center, W_context):
    B, NNEG = negatives.shape
    D = W_center.shape[1]
    pos, negt = _sc_scores(B, NNEG, D,
                           center.astype(jnp.int32),
                           context.astype(jnp.int32),
                           negatives.reshape(-1).astype(jnp.int32),
                           W_center, W_context)
    loss = pl.pallas_call(
        functools.partial(_loss_kernel, inv_b=1.0 / B),
        out_shape=jax.ShapeDtypeStruct((1, 1), jnp.float32),
    )(pos.reshape(B // 128, 128), negt.reshape(NNEG * B // 128, 128))
    return loss[0, 0]
